# baseline (speedup 1.0000x reference)
import functools

import jax
import jax.numpy as jnp
from jax import lax
from jax.experimental import pallas as pl
from jax.experimental.pallas import tpu as pltpu
from jax.experimental.pallas import tpu_sc as plsc

NUM_E = 64
N_TOK = 8192
D = 768
TM = 128
NB = N_TOK // TM
MAX_ITEMS = NB + NUM_E - 1

NC = 2
NS = 16
NW = NC * NS
ROWS_W = N_TOK // NW
CHUNK = 64
NCHUNK = ROWS_W // CHUNK

def _sc_mesh():
    return plsc.VectorSubcoreMesh(
        core_axis_name="c", subcore_axis_name="s",
        num_cores=NC, num_subcores=NS)


@functools.cache
def _gather_kernel():
    @functools.partial(
        pl.kernel,
        out_type=jax.ShapeDtypeStruct((N_TOK, D), jnp.float32),
        mesh=_sc_mesh(),
        scratch_types=[
            pltpu.VMEM((CHUNK,), jnp.int32),
            pltpu.VMEM((CHUNK, D), jnp.float32),
            pltpu.SemaphoreType.DMA,
        ],
    )
    def _gather_k(h_hbm, idx_hbm, xs_hbm, idx_c, rows, sem):
        wid = lax.axis_index("s") * NC + lax.axis_index("c")
        base = wid * ROWS_W
        for c in range(NCHUNK):
            pltpu.sync_copy(idx_hbm.at[pl.ds(base + c * CHUNK, CHUNK)], idx_c)
            pltpu.async_copy(h_hbm.at[idx_c], rows, sem).wait()
            pltpu.sync_copy(rows, xs_hbm.at[pl.ds(base + c * CHUNK, CHUNK)])

    return _gather_k


@functools.cache
def _scatter_kernel():
    @functools.partial(
        pl.kernel,
        out_type=jax.ShapeDtypeStruct((N_TOK, D), jnp.float32),
        mesh=_sc_mesh(),
        scratch_types=[
            pltpu.VMEM((CHUNK,), jnp.int32),
            pltpu.VMEM((CHUNK, D), jnp.float32),
            pltpu.SemaphoreType.DMA,
        ],
    )
    def _scatter_k(y_hbm, idx_hbm, out_hbm, idx_c, rows, sem):
        wid = lax.axis_index("s") * NC + lax.axis_index("c")
        base = wid * ROWS_W
        for c in range(NCHUNK):
            pltpu.sync_copy(idx_hbm.at[pl.ds(base + c * CHUNK, CHUNK)], idx_c)
            pltpu.sync_copy(y_hbm.at[pl.ds(base + c * CHUNK, CHUNK)], rows)
            pltpu.async_copy(rows, out_hbm.at[idx_c], sem).wait()

    return _scatter_k


def _mm_body(tab_r, blk_r, x_r, w_hbm, b_r, rw_r, o_r, wf, wb, wsem):
    bidx = pl.program_id(0)

    @pl.when(bidx == 0)
    def _():
        e0 = tab_r[0, 0]
        pltpu.make_async_copy(
            w_hbm.at[pl.ds(e0, 1)], wf.at[pl.ds(0, 1)], wsem).start()

    i0 = blk_r[0, bidx]
    i1 = blk_r[1, bidx]
    rows = bidx * TM + lax.broadcasted_iota(jnp.int32, (TM, 1), 0)
    xb = x_r[...].astype(jnp.bfloat16)
    rwcol = rw_r[...][:, :1]

    def body(i, carry):
        eid = tab_r[0, i]
        st = tab_r[1, i]
        en = tab_r[2, i]
        wch = tab_r[3, i]
        wsl = tab_r[4, i]
        wis = tab_r[5, i]
        wie = tab_r[6, i]
        wisl = tab_r[7, i]

        @pl.when(wch == 1)
        def _():
            pltpu.make_async_copy(
                w_hbm.at[pl.ds(eid, 1)], wf.at[pl.ds(wsl, 1)], wsem).wait()
            wb[...] = wf[pl.ds(wsl, 1)][0].astype(jnp.bfloat16)

        @pl.when(wis == 1)
        def _():
            pltpu.make_async_copy(
                w_hbm.at[pl.ds(wie, 1)], wf.at[pl.ds(wisl, 1)], wsem).start()

        acc = jnp.dot(xb, wb[...], preferred_element_type=jnp.float32)
        acc = (acc + b_r[pl.ds(eid, 1)][0]) * rwcol
        mask = (rows >= st) & (rows < en)
        prev = jnp.where(i == i0, jnp.zeros_like(acc), o_r[...])
        o_r[...] = jnp.where(mask, acc, prev)
        return carry

    lax.fori_loop(i0, i1, body, 0)


def _grouped_matmul(xs, W, b3, rws2, tab, blk):
    return pl.pallas_call(
        _mm_body,
        grid=(NB,),
        in_specs=[
            pl.BlockSpec(memory_space=pltpu.MemorySpace.SMEM),
            pl.BlockSpec(memory_space=pltpu.MemorySpace.SMEM),
            pl.BlockSpec((TM, D), lambda i: (i, 0)),
            pl.BlockSpec(memory_space=pltpu.MemorySpace.HBM),
            pl.BlockSpec((NUM_E, 1, D), lambda i: (0, 0, 0)),
            pl.BlockSpec((TM, 2), lambda i: (i, 0)),
        ],
        out_specs=pl.BlockSpec((TM, D), lambda i: (i, 0)),
        out_shape=jax.ShapeDtypeStruct((N_TOK, D), jnp.float32),
        scratch_shapes=[
            pltpu.VMEM((2, D, D), jnp.float32),
            pltpu.VMEM((D, D), jnp.bfloat16),
            pltpu.SemaphoreType.DMA,
        ],
        compiler_params=pltpu.CompilerParams(
            dimension_semantics=("arbitrary",)),
    )(tab, blk, xs, W, b3, rws2)


def _work_items(primary):
    counts = jnp.bincount(primary, length=NUM_E)
    ends = jnp.cumsum(counts)
    starts = ends - counts
    fb = starts // TM
    lb = (ends - 1) // TM
    nb = jnp.where(counts > 0, lb - fb + 1, 0)
    cum_nb = jnp.cumsum(nb)
    excl = cum_nb - nb
    total = cum_nb[-1]
    ii_raw = jnp.arange(MAX_ITEMS, dtype=jnp.int32)
    ii = jnp.minimum(ii_raw, total - 1)
    eid = jnp.searchsorted(cum_nb, ii, side="right").astype(jnp.int32)
    bid = (fb[eid] + ii - excl[eid]).astype(jnp.int32)
    st = starts[eid].astype(jnp.int32)
    en = ends[eid].astype(jnp.int32)
    bid_s = jnp.where(ii_raw < total, bid, NB)
    blocks = jnp.arange(NB, dtype=jnp.int32)
    istart = jnp.searchsorted(bid_s, blocks, side="left").astype(jnp.int32)
    iend = jnp.searchsorted(bid_s, blocks, side="right").astype(jnp.int32)
    wchg = jnp.concatenate(
        [jnp.ones((1,), jnp.int32), (eid[1:] != eid[:-1]).astype(jnp.int32)])
    wcount = jnp.cumsum(wchg)
    wslot = (wcount - 1) % 2
    nloads = jnp.take(wcount, total - 1)
    load_eid = jnp.zeros((MAX_ITEMS + 1,), jnp.int32).at[wcount - 1].set(eid)
    wissue = (wchg == 1) & (wcount < nloads)
    wissue_eid = load_eid[wcount]
    wissue_slot = wcount % 2
    tab = jnp.stack([eid, st, en, wchg, wslot,
                     wissue.astype(jnp.int32), wissue_eid, wissue_slot])
    blk = jnp.stack([istart, iend])
    return tab, blk


def kernel(hidden_states, expert_indices, routing_weights, W, b):
    primary = expert_indices[:, 0].astype(jnp.int32)
    sorted_idx = jnp.argsort(primary, stable=True).astype(jnp.int32)
    tab, blk = _work_items(primary)
    xs = _gather_kernel()(hidden_states, sorted_idx)
    rws2 = jnp.take(routing_weights, sorted_idx, axis=0)
    y = _grouped_matmul(xs, W, b[:, None, :], rws2, tab, blk)
    return _scatter_kernel()(y, sorted_idx)
